# baseline (speedup 1.0000x reference)
import functools

import jax
import jax.numpy as jnp
from jax import lax
from jax.experimental import pallas as pl
from jax.experimental.pallas import tpu as pltpu
from jax.experimental.pallas import tpu_sc as plsc

B, T, F = 16, 2048, 1024
E = 4
TCB = B - E
W_PER_EX = 8
ROWS_PER_W = T // W_PER_EX
CH = 32
K = ROWS_PER_W // CH
NVREG = F // 16


def _tc_body(x_ref, o_ref):
    b = pl.program_id(0)
    x = x_ref[0]
    m = jnp.max(jnp.abs(x), axis=1)
    c = jnp.sum((m > 0.0).astype(jnp.int32))
    t = jnp.maximum(c - 1, 0)
    o_ref[pl.ds(b, 1), :] = x_ref[0, pl.ds(t, 1), :]


_tc_part = pl.pallas_call(
    _tc_body,
    grid=(TCB,),
    in_specs=[pl.BlockSpec((1, T, F), lambda b: (b, 0, 0))],
    out_specs=pl.BlockSpec((TCB, F), lambda b: (0, 0)),
    out_shape=jax.ShapeDtypeStruct((TCB, F), jnp.float32),
    compiler_params=pltpu.CompilerParams(
        dimension_semantics=("arbitrary",),
    ),
)


_sc_mesh = plsc.VectorSubcoreMesh(core_axis_name="c", subcore_axis_name="s", num_cores=2)


@functools.partial(
    pl.kernel,
    mesh=_sc_mesh,
    out_type=jax.ShapeDtypeStruct((E, F), jnp.float32),
    scratch_types=[
        pltpu.VMEM((2, CH, F), jnp.float32),
        pltpu.VMEM((16,), jnp.int32),
        pltpu.VMEM((8, 16), jnp.int32),
        pltpu.VMEM((1, F), jnp.float32),
        pltpu.VMEM_SHARED((16, 16), jnp.int32),
        pltpu.SemaphoreType.DMA,
        pltpu.SemaphoreType.DMA,
    ],
    compiler_params=pltpu.CompilerParams(needs_layout_passes=False),
)
def _sc_part(x_hbm, out_hbm, buf, stage, cntb, row, shared, sem0, sem1):
    c = lax.axis_index("c")
    s = lax.axis_index("s")
    n_local = 16 // W_PER_EX
    local_e = s // W_PER_EX
    e_sc = c * n_local + local_e
    base_row = (TCB + e_sc) * T + (s % W_PER_EX) * ROWS_PER_W

    sems = (sem0, sem1)

    def copy_chunk(k_chunk, slot):
        return pltpu.make_async_copy(
            x_hbm.at[pl.ds(base_row + k_chunk * CH, CH)],
            buf.at[slot],
            sems[slot],
        )

    copy_chunk(0, 0).start()
    copy_chunk(1, 1).start()

    one16 = jnp.ones((16,), jnp.int32)

    def count_chunk(slot, cnt0):
        def row_body(r, cnt):
            accs = [jnp.abs(buf[slot, r, pl.ds(16 * a, 16)]) for a in range(4)]
            for j in range(4, NVREG):
                a = j % 4
                accs[a] = jnp.maximum(
                    accs[a], jnp.abs(buf[slot, r, pl.ds(16 * j, 16)])
                )
            acc = jnp.maximum(
                jnp.maximum(accs[0], accs[1]), jnp.maximum(accs[2], accs[3])
            )
            pop = plsc.all_reduce_population_count(acc > 0.0)
            return cnt + jnp.minimum(pop, one16)

        return lax.fori_loop(0, CH, row_body, cnt0)

    def outer(ko, cnt):
        for slot in range(2):
            k_chunk = 2 * ko + slot
            copy_chunk(k_chunk, slot).wait()
            cnt = count_chunk(slot, cnt)

            @pl.when(k_chunk + 2 < K)
            def _():
                copy_chunk(k_chunk + 2, slot).start()
        return cnt

    cnt_vec = lax.fori_loop(0, K // 2, outer, jnp.zeros((16,), jnp.int32))

    stage[...] = cnt_vec
    pltpu.sync_copy(stage, shared.at[s])
    plsc.subcore_barrier()

    @pl.when(s % W_PER_EX == 0)
    def _():
        pltpu.sync_copy(shared.at[pl.ds(s, W_PER_EX)], cntb)
        total_vec = cntb[0, :]
        for r in range(1, W_PER_EX):
            total_vec = total_vec + cntb[r, :]
        t_vec = jnp.minimum(jnp.maximum(total_vec - 1, 0), T - 1)
        t = t_vec[0]
        g = (TCB + e_sc) * T + t
        pltpu.sync_copy(x_hbm.at[pl.ds(g, 1)], row)
        pltpu.sync_copy(row, out_hbm.at[pl.ds(e_sc, 1)])


def kernel(inputs):
    x2d = inputs.reshape(B * T, F)
    sc_out = _sc_part(x2d)
    tc_out = _tc_part(inputs)
    return jnp.concatenate([tc_out, sc_out], axis=0)


# speedup vs baseline: 1.0241x; 1.0241x over previous
import functools

import jax
import jax.numpy as jnp
from jax import lax
from jax.experimental import pallas as pl
from jax.experimental.pallas import tpu as pltpu
from jax.experimental.pallas import tpu_sc as plsc

B, T, F = 16, 2048, 1024
E = 2
TCB = B - E
W_PER_EX = 16
ROWS_PER_W = T // W_PER_EX
CH = 32
K = ROWS_PER_W // CH
NVREG = F // 16


def _tc_body(x_ref, o_ref):
    b = pl.program_id(0)
    x = x_ref[0]
    m = jnp.max(jnp.abs(x), axis=1)
    c = jnp.sum((m > 0.0).astype(jnp.int32))
    t = jnp.maximum(c - 1, 0)
    o_ref[pl.ds(b, 1), :] = x_ref[0, pl.ds(t, 1), :]


_tc_part = pl.pallas_call(
    _tc_body,
    grid=(TCB,),
    in_specs=[pl.BlockSpec((1, T, F), lambda b: (b, 0, 0))],
    out_specs=pl.BlockSpec((TCB, F), lambda b: (0, 0)),
    out_shape=jax.ShapeDtypeStruct((TCB, F), jnp.float32),
    compiler_params=pltpu.CompilerParams(
        dimension_semantics=("arbitrary",),
    ),
)


_sc_mesh = plsc.VectorSubcoreMesh(core_axis_name="c", subcore_axis_name="s", num_cores=2)


@functools.partial(
    pl.kernel,
    mesh=_sc_mesh,
    out_type=jax.ShapeDtypeStruct((E, F), jnp.float32),
    scratch_types=[
        pltpu.VMEM((2, CH, F), jnp.float32),
        pltpu.VMEM((16,), jnp.int32),
        pltpu.VMEM((W_PER_EX, 16), jnp.int32),
        pltpu.VMEM((1, F), jnp.float32),
        pltpu.VMEM_SHARED((16, 16), jnp.int32),
        pltpu.SemaphoreType.DMA,
        pltpu.SemaphoreType.DMA,
    ],
    compiler_params=pltpu.CompilerParams(needs_layout_passes=False),
)
def _sc_part(x_hbm, out_hbm, buf, stage, cntb, row, shared, sem0, sem1):
    c = lax.axis_index("c")
    s = lax.axis_index("s")
    n_local = 16 // W_PER_EX
    local_e = s // W_PER_EX
    e_sc = c * n_local + local_e
    base_row = (TCB + e_sc) * T + (s % W_PER_EX) * ROWS_PER_W

    sems = (sem0, sem1)

    def copy_chunk(k_chunk, slot):
        return pltpu.make_async_copy(
            x_hbm.at[pl.ds(base_row + k_chunk * CH, CH)],
            buf.at[slot],
            sems[slot],
        )

    copy_chunk(0, 0).start()
    copy_chunk(1, 1).start()

    one16 = jnp.ones((16,), jnp.int32)

    def count_chunk(slot, cnt0):
        def row_body(r, cnt):
            accs = [jnp.abs(buf[slot, r, pl.ds(16 * a, 16)]) for a in range(4)]
            for j in range(4, NVREG):
                a = j % 4
                accs[a] = jnp.maximum(
                    accs[a], jnp.abs(buf[slot, r, pl.ds(16 * j, 16)])
                )
            acc = jnp.maximum(
                jnp.maximum(accs[0], accs[1]), jnp.maximum(accs[2], accs[3])
            )
            pop = plsc.all_reduce_population_count(acc > 0.0)
            return cnt + jnp.minimum(pop, one16)

        return lax.fori_loop(0, CH, row_body, cnt0)

    def outer(ko, cnt):
        for slot in range(2):
            k_chunk = 2 * ko + slot
            copy_chunk(k_chunk, slot).wait()
            cnt = count_chunk(slot, cnt)

            @pl.when(k_chunk + 2 < K)
            def _():
                copy_chunk(k_chunk + 2, slot).start()
        return cnt

    cnt_vec = lax.fori_loop(0, K // 2, outer, jnp.zeros((16,), jnp.int32))

    stage[...] = cnt_vec
    pltpu.sync_copy(stage, shared.at[s])
    plsc.subcore_barrier()

    @pl.when(s % W_PER_EX == 0)
    def _():
        pltpu.sync_copy(shared.at[pl.ds(s, W_PER_EX)], cntb)
        total_vec = cntb[0, :]
        for r in range(1, W_PER_EX):
            total_vec = total_vec + cntb[r, :]
        t_vec = jnp.minimum(jnp.maximum(total_vec - 1, 0), T - 1)
        t = t_vec[0]
        g = (TCB + e_sc) * T + t
        pltpu.sync_copy(x_hbm.at[pl.ds(g, 1)], row)
        pltpu.sync_copy(row, out_hbm.at[pl.ds(e_sc, 1)])


def kernel(inputs):
    x2d = inputs.reshape(B * T, F)
    sc_out = _sc_part(x2d)
    tc_out = _tc_part(inputs)
    return jnp.concatenate([tc_out, sc_out], axis=0)


# speedup vs baseline: 1.4413x; 1.4074x over previous
import jax
import jax.numpy as jnp
from jax.experimental import pallas as pl
from jax.experimental.pallas import tpu as pltpu

B, T, F = 16, 2048, 1024


def _body(x_ref, o_ref):
    b = pl.program_id(0)
    x = x_ref[0]
    m = jnp.max(jnp.abs(x), axis=1)
    c = jnp.sum((m > 0.0).astype(jnp.int32))
    t = jnp.maximum(c - 1, 0)
    o_ref[pl.ds(b, 1), :] = x_ref[0, pl.ds(t, 1), :]


_fused = pl.pallas_call(
    _body,
    grid=(B,),
    in_specs=[pl.BlockSpec((1, T, F), lambda b: (b, 0, 0))],
    out_specs=pl.BlockSpec((B, F), lambda b: (0, 0)),
    out_shape=jax.ShapeDtypeStruct((B, F), jnp.float32),
    compiler_params=pltpu.CompilerParams(
        dimension_semantics=("arbitrary",),
    ),
)


def kernel(inputs):
    return _fused(inputs)
